# baseline (speedup 1.0000x reference)
import functools

import jax
import jax.numpy as jnp
from jax import lax
from jax.experimental import pallas as pl
from jax.experimental.pallas import tpu as pltpu
from jax.experimental.pallas import tpu_sc as plsc

NC = 2
NS = 16
NW = NC * NS
CH = 128
L = 16


def _gather_call(ids2d, wordtab, n_per_w, n_ch):
    mesh = plsc.VectorSubcoreMesh(core_axis_name="c", subcore_axis_name="s")
    N = NW * n_per_w

    @functools.partial(
        pl.kernel,
        mesh=mesh,
        out_type=jax.ShapeDtypeStruct((N, 32), jnp.int32),
        scratch_types=[
            pltpu.VMEM((n_ch, CH), jnp.int32),
            pltpu.VMEM((n_ch, CH), jnp.int32),
            pltpu.VMEM((n_per_w, 128), jnp.int32),
            pltpu.VMEM((n_per_w, 32), jnp.int32),
            pltpu.SemaphoreType.DMA,
        ],
        compiler_params=pltpu.CompilerParams(
            use_tc_tiling_on_sc=False, needs_layout_passes=False
        ),
    )
    def gather_kernel(ids_hbm, tab_hbm, out_hbm, idx_v, q_v, rows_v, sel_v, sem):
        wid = lax.axis_index("s") * NC + lax.axis_index("c")
        base = wid * n_per_w
        pltpu.sync_copy(ids_hbm.at[pl.ds(wid * n_ch, n_ch)], idx_v)

        def qbody(i, _):
            j, k = i // (CH // L), (i % (CH // L)) * L
            vec = idx_v[j, pl.ds(k, L)]
            q_v[j, pl.ds(k, L)] = lax.shift_right_logical(vec, 2)
            return 0

        lax.fori_loop(0, n_ch * (CH // L), qbody, 0, unroll=True)

        copies = []
        for j in range(n_ch):
            copies.append(
                pltpu.async_copy(
                    tab_hbm.at[q_v.at[j]],
                    rows_v.at[pl.ds(j * CH, CH)],
                    sem,
                )
            )
        for c in copies:
            c.wait()

        lanes = lax.broadcasted_iota(jnp.int32, (L,), 0)

        def sbody(g, _):
            j, k0 = g // (CH // L), (g % (CH // L)) * L
            idxv = idx_v[j, pl.ds(k0, L)]
            rowv = j * CH + k0 + lanes
            col0 = lax.shift_left(lax.bitwise_and(idxv, 3), 5)

            def wbody(k, _):
                w = plsc.load_gather(rows_v, [rowv, col0 + k])
                plsc.store_scatter(sel_v, [rowv, lax.full((L,), 0, jnp.int32) + k], w)
                return 0

            lax.fori_loop(0, 32, wbody, 0, unroll=8)
            return 0

        lax.fori_loop(0, n_per_w // L, sbody, 0)
        pltpu.sync_copy(sel_v, out_hbm.at[pl.ds(base, n_per_w)])

    return gather_kernel(ids2d, wordtab)


def kernel(input_ids, wte):
    B, S = input_ids.shape
    V, D = wte.shape
    N = B * S
    n_per_w = N // NW
    n_ch = n_per_w // CH
    ids2d = input_ids.reshape(NW * n_ch, CH)
    wordtab = lax.bitcast_convert_type(
        wte.reshape(V * D // 256, 128, 2), jnp.int32
    )
    out32 = _gather_call(ids2d, wordtab, n_per_w, n_ch)
    out = lax.bitcast_convert_type(out32, jnp.bfloat16).reshape(B, S, D)
    return out.astype(jnp.float32)


# speedup vs baseline: 17.5498x; 17.5498x over previous
import functools

import jax
import jax.numpy as jnp
from jax import lax
from jax.experimental import pallas as pl
from jax.experimental.pallas import tpu as pltpu
from jax.experimental.pallas import tpu_sc as plsc


def _format_words(wte_t):
    D, V = wte_t.shape
    blk = 512
    grid = (V + blk - 1) // blk
    nq = blk * D // 256

    def body(in_ref, out_ref):
        x = in_ref[...]
        xi = pltpu.bitcast(x, jnp.int32)
        wt = xi.T
        w4 = wt.reshape(nq, 4, 32)
        out_ref[...] = jnp.concatenate(
            [w4[:, a, :] for a in range(4)], axis=1
        )

    return pl.pallas_call(
        body,
        grid=(grid,),
        in_specs=[pl.BlockSpec((D, blk), lambda i: (0, i))],
        out_specs=pl.BlockSpec((nq, 128), lambda i: (i, 0)),
        out_shape=jax.ShapeDtypeStruct((V * D // 256, 128), jnp.int32),
    )(wte_t)

NC = 2
NS = 16
NW = NC * NS
CH = 128
L = 16


def _gather_call(ids2d, wordtab, n_per_w, n_ch):
    mesh = plsc.VectorSubcoreMesh(core_axis_name="c", subcore_axis_name="s")
    N = NW * n_per_w

    @functools.partial(
        pl.kernel,
        mesh=mesh,
        out_type=jax.ShapeDtypeStruct((N, 32), jnp.int32),
        scratch_types=[
            pltpu.VMEM((n_ch, CH), jnp.int32),
            pltpu.VMEM((n_ch, CH), jnp.int32),
            pltpu.VMEM((n_per_w, 128), jnp.int32),
            pltpu.VMEM((n_per_w, 32), jnp.int32),
            pltpu.SemaphoreType.DMA,
        ],
        compiler_params=pltpu.CompilerParams(
            use_tc_tiling_on_sc=False, needs_layout_passes=False
        ),
    )
    def gather_kernel(ids_hbm, tab_hbm, out_hbm, idx_v, q_v, rows_v, sel_v, sem):
        wid = lax.axis_index("s") * NC + lax.axis_index("c")
        base = wid * n_per_w
        pltpu.sync_copy(ids_hbm.at[pl.ds(wid * n_ch, n_ch)], idx_v)

        def qbody(i, _):
            j, k = i // (CH // L), (i % (CH // L)) * L
            vec = idx_v[j, pl.ds(k, L)]
            q_v[j, pl.ds(k, L)] = lax.shift_right_logical(vec, 2)
            return 0

        lax.fori_loop(0, n_ch * (CH // L), qbody, 0, unroll=True)

        copies = []
        for j in range(n_ch):
            copies.append(
                pltpu.async_copy(
                    tab_hbm.at[q_v.at[j]],
                    rows_v.at[pl.ds(j * CH, CH)],
                    sem,
                )
            )
        for c in copies:
            c.wait()

        lanes = lax.broadcasted_iota(jnp.int32, (L,), 0)

        def sbody(g, _):
            j, k0 = g // (CH // L), (g % (CH // L)) * L
            idxv = idx_v[j, pl.ds(k0, L)]
            rowv = j * CH + k0 + lanes
            col0 = lax.shift_left(lax.bitwise_and(idxv, 3), 5)

            def wbody(k, _):
                w = plsc.load_gather(rows_v, [rowv, col0 + k])
                plsc.store_scatter(sel_v, [rowv, lax.full((L,), 0, jnp.int32) + k], w)
                return 0

            lax.fori_loop(0, 32, wbody, 0, unroll=8)
            return 0

        lax.fori_loop(0, n_per_w // L, sbody, 0)
        pltpu.sync_copy(sel_v, out_hbm.at[pl.ds(base, n_per_w)])

    return gather_kernel(ids2d, wordtab)


def kernel(input_ids, wte):
    B, S = input_ids.shape
    V, D = wte.shape
    N = B * S
    n_per_w = N // NW
    n_ch = n_per_w // CH
    ids2d = input_ids.reshape(NW * n_ch, CH)
    wordtab = _format_words(wte.T)
    out32 = _gather_call(ids2d, wordtab, n_per_w, n_ch)
    out = lax.bitcast_convert_type(out32, jnp.bfloat16).reshape(B, S, D)
    return out.astype(jnp.float32)


# speedup vs baseline: 38.5398x; 2.1960x over previous
import functools

import jax
import jax.numpy as jnp
from jax import lax
from jax.experimental import pallas as pl
from jax.experimental.pallas import tpu as pltpu
from jax.experimental.pallas import tpu_sc as plsc


def _format_words(wte_t):
    D, V = wte_t.shape
    blk = 2048
    grid = (V + blk - 1) // blk
    nq = blk * D // 256

    dk = D // 2

    def body(in_ref, out_ref):
        ks = lax.broadcasted_iota(jnp.int32, (D, dk), 1)
        ds_ = lax.broadcasted_iota(jnp.int32, (D, dk), 0)
        p_even = (ds_ == 2 * ks).astype(jnp.bfloat16)
        p_odd = (ds_ == 2 * ks + 1).astype(jnp.bfloat16)
        x = in_ref[...]
        dn = (((0,), (0,)), ((), ()))
        ye = lax.dot_general(x, p_even, dn, preferred_element_type=jnp.float32)
        yo = lax.dot_general(x, p_odd, dn, preferred_element_type=jnp.float32)
        be = lax.bitcast_convert_type(ye, jnp.int32)
        bo = lax.bitcast_convert_type(yo, jnp.int32)
        w = lax.bitwise_or(
            lax.shift_right_logical(be, 16),
            lax.bitwise_and(bo, jnp.int32(-65536)),
        )
        w4 = w.reshape(nq, 4, dk)
        out_ref[...] = jnp.concatenate(
            [w4[:, a, :] for a in range(4)], axis=1
        )

    return pl.pallas_call(
        body,
        grid=(grid,),
        in_specs=[pl.BlockSpec((D, blk), lambda i: (0, i))],
        out_specs=pl.BlockSpec((nq, 128), lambda i: (i, 0)),
        out_shape=jax.ShapeDtypeStruct((V * D // 256, 128), jnp.int32),
    )(wte_t)

NC = 2
NS = 16
NW = NC * NS
CH = 128
L = 16


def _gather_call(ids2d, wordtab, n_per_w, n_ch):
    mesh = plsc.VectorSubcoreMesh(core_axis_name="c", subcore_axis_name="s")
    N = NW * n_per_w

    @functools.partial(
        pl.kernel,
        mesh=mesh,
        out_type=jax.ShapeDtypeStruct((N, 32), jnp.int32),
        scratch_types=[
            pltpu.VMEM((n_ch, CH), jnp.int32),
            pltpu.VMEM((n_ch, CH), jnp.int32),
            pltpu.VMEM((n_per_w, 128), jnp.int32),
            pltpu.VMEM((n_per_w, 32), jnp.int32),
            pltpu.SemaphoreType.DMA,
        ],
        compiler_params=pltpu.CompilerParams(
            use_tc_tiling_on_sc=False, needs_layout_passes=False
        ),
    )
    def gather_kernel(ids_hbm, tab_hbm, out_hbm, idx_v, q_v, rows_v, sel_v, sem):
        wid = lax.axis_index("s") * NC + lax.axis_index("c")
        base = wid * n_per_w
        pltpu.sync_copy(ids_hbm.at[pl.ds(wid * n_ch, n_ch)], idx_v)

        def qbody(i, _):
            j, k = i // (CH // L), (i % (CH // L)) * L
            vec = idx_v[j, pl.ds(k, L)]
            q_v[j, pl.ds(k, L)] = lax.shift_right_logical(vec, 2)
            return 0

        lax.fori_loop(0, n_ch * (CH // L), qbody, 0, unroll=True)

        copies = []
        for j in range(n_ch):
            copies.append(
                pltpu.async_copy(
                    tab_hbm.at[q_v.at[j]],
                    rows_v.at[pl.ds(j * CH, CH)],
                    sem,
                )
            )
        for c in copies:
            c.wait()

        lanes = lax.broadcasted_iota(jnp.int32, (L,), 0)

        def sbody(g, _):
            j, k0 = g // (CH // L), (g % (CH // L)) * L
            idxv = idx_v[j, pl.ds(k0, L)]
            rowv = j * CH + k0 + lanes
            col0 = lax.shift_left(lax.bitwise_and(idxv, 3), 5)

            def wbody(k, _):
                w = plsc.load_gather(rows_v, [rowv, col0 + k])
                plsc.store_scatter(sel_v, [rowv, lax.full((L,), 0, jnp.int32) + k], w)
                return 0

            lax.fori_loop(0, 32, wbody, 0, unroll=8)
            return 0

        lax.fori_loop(0, n_per_w // L, sbody, 0)
        pltpu.sync_copy(sel_v, out_hbm.at[pl.ds(base, n_per_w)])

    return gather_kernel(ids2d, wordtab)


def kernel(input_ids, wte):
    B, S = input_ids.shape
    V, D = wte.shape
    N = B * S
    n_per_w = N // NW
    n_ch = n_per_w // CH
    ids2d = input_ids.reshape(NW * n_ch, CH)
    wordtab = _format_words(wte.T)
    out32 = _gather_call(ids2d, wordtab, n_per_w, n_ch)
    out = lax.bitcast_convert_type(out32, jnp.bfloat16).reshape(B, S, D)
    return out.astype(jnp.float32)


# speedup vs baseline: 44.0807x; 1.1438x over previous
import functools

import jax
import jax.numpy as jnp
from jax import lax
from jax.experimental import pallas as pl
from jax.experimental.pallas import tpu as pltpu
from jax.experimental.pallas import tpu_sc as plsc


def _format_words(wte_t):
    D, V = wte_t.shape
    blk = 4096
    grid = (V + blk - 1) // blk
    nq = blk * D // 256

    dk = D // 2

    def body(in_ref, out_ref):
        ks = lax.broadcasted_iota(jnp.int32, (D, dk), 1)
        ds_ = lax.broadcasted_iota(jnp.int32, (D, dk), 0)
        p_even = (ds_ == 2 * ks).astype(jnp.bfloat16)
        p_odd = (ds_ == 2 * ks + 1).astype(jnp.bfloat16)
        x = in_ref[...]
        dn = (((0,), (0,)), ((), ()))
        ye = lax.dot_general(x, p_even, dn, preferred_element_type=jnp.float32)
        yo = lax.dot_general(x, p_odd, dn, preferred_element_type=jnp.float32)
        be = lax.bitcast_convert_type(ye, jnp.int32)
        bo = lax.bitcast_convert_type(yo, jnp.int32)
        w = lax.bitwise_or(
            lax.shift_right_logical(be, 16),
            lax.bitwise_and(bo, jnp.int32(-65536)),
        )
        w4 = w.reshape(nq, 4, dk)
        out_ref[...] = jnp.concatenate(
            [w4[:, a, :] for a in range(4)], axis=1
        )

    return pl.pallas_call(
        body,
        grid=(grid,),
        in_specs=[pl.BlockSpec((D, blk), lambda i: (0, i))],
        out_specs=pl.BlockSpec((nq, 128), lambda i: (i, 0)),
        out_shape=jax.ShapeDtypeStruct((V * D // 256, 128), jnp.int32),
    )(wte_t)

NC = 2
NS = 16
NW = NC * NS
CH = 128
L = 16


def _gather_call(ids2d, wordtab, n_per_w, n_ch):
    mesh = plsc.VectorSubcoreMesh(core_axis_name="c", subcore_axis_name="s")
    N = NW * n_per_w

    @functools.partial(
        pl.kernel,
        mesh=mesh,
        out_type=jax.ShapeDtypeStruct((N, 32), jnp.int32),
        scratch_types=[
            pltpu.VMEM((n_ch, CH), jnp.int32),
            pltpu.VMEM((n_ch, CH), jnp.int32),
            pltpu.VMEM((n_per_w, 128), jnp.int32),
            pltpu.VMEM((n_per_w, 32), jnp.int32),
            pltpu.SemaphoreType.DMA,
        ],
        compiler_params=pltpu.CompilerParams(
            use_tc_tiling_on_sc=False, needs_layout_passes=False
        ),
    )
    def gather_kernel(ids_hbm, tab_hbm, out_hbm, idx_v, q_v, rows_v, sel_v, sem):
        wid = lax.axis_index("s") * NC + lax.axis_index("c")
        base = wid * n_per_w
        pltpu.sync_copy(ids_hbm.at[pl.ds(wid * n_ch, n_ch)], idx_v)

        def qbody(i, _):
            j, k = i // (CH // L), (i % (CH // L)) * L
            vec = idx_v[j, pl.ds(k, L)]
            q_v[j, pl.ds(k, L)] = lax.shift_right_logical(vec, 2)
            return 0

        lax.fori_loop(0, n_ch * (CH // L), qbody, 0, unroll=True)

        copies = []
        for j in range(n_ch):
            copies.append(
                pltpu.async_copy(
                    tab_hbm.at[q_v.at[j]],
                    rows_v.at[pl.ds(j * CH, CH)],
                    sem,
                )
            )
        for c in copies:
            c.wait()

        lanes = lax.broadcasted_iota(jnp.int32, (L,), 0)

        def sbody(g, _):
            j, k0 = g // (CH // L), (g % (CH // L)) * L
            idxv = idx_v[j, pl.ds(k0, L)]
            rowv = j * CH + k0 + lanes
            col0 = lax.shift_left(lax.bitwise_and(idxv, 3), 5)

            def wbody(k, _):
                w = plsc.load_gather(rows_v, [rowv, col0 + k])
                plsc.store_scatter(sel_v, [rowv, lax.full((L,), 0, jnp.int32) + k], w)
                return 0

            lax.fori_loop(0, 32, wbody, 0, unroll=8)
            return 0

        lax.fori_loop(0, n_per_w // L, sbody, 0)
        pltpu.sync_copy(sel_v, out_hbm.at[pl.ds(base, n_per_w)])

    return gather_kernel(ids2d, wordtab)


def kernel(input_ids, wte):
    B, S = input_ids.shape
    V, D = wte.shape
    N = B * S
    n_per_w = N // NW
    n_ch = n_per_w // CH
    ids2d = input_ids.reshape(NW * n_ch, CH)
    wordtab = _format_words(wte.T)
    out32 = _gather_call(ids2d, wordtab, n_per_w, n_ch)
    out = lax.bitcast_convert_type(out32, jnp.bfloat16).reshape(B, S, D)
    return out.astype(jnp.float32)


# speedup vs baseline: 44.6364x; 1.0126x over previous
import functools

import jax
import jax.numpy as jnp
from jax import lax
from jax.experimental import pallas as pl
from jax.experimental.pallas import tpu as pltpu
from jax.experimental.pallas import tpu_sc as plsc


def _format_words(wte_t):
    D, V = wte_t.shape
    blk = 8192
    grid = (V + blk - 1) // blk
    nq = blk * D // 256

    dk = D // 2

    def body(in_ref, out_ref):
        ks = lax.broadcasted_iota(jnp.int32, (D, dk), 1)
        ds_ = lax.broadcasted_iota(jnp.int32, (D, dk), 0)
        p_even = (ds_ == 2 * ks).astype(jnp.bfloat16)
        p_odd = (ds_ == 2 * ks + 1).astype(jnp.bfloat16)
        x = in_ref[...]
        dn = (((0,), (0,)), ((), ()))
        ye = lax.dot_general(x, p_even, dn, preferred_element_type=jnp.float32)
        yo = lax.dot_general(x, p_odd, dn, preferred_element_type=jnp.float32)
        be = lax.bitcast_convert_type(ye, jnp.int32)
        bo = lax.bitcast_convert_type(yo, jnp.int32)
        w = lax.bitwise_or(
            lax.shift_right_logical(be, 16),
            lax.bitwise_and(bo, jnp.int32(-65536)),
        )
        w4 = w.reshape(nq, 4, dk)
        out_ref[...] = jnp.concatenate(
            [w4[:, a, :] for a in range(4)], axis=1
        )

    return pl.pallas_call(
        body,
        grid=(grid,),
        in_specs=[pl.BlockSpec((D, blk), lambda i: (0, i))],
        out_specs=pl.BlockSpec((nq, 128), lambda i: (i, 0)),
        out_shape=jax.ShapeDtypeStruct((V * D // 256, 128), jnp.int32),
    )(wte_t)

NC = 2
NS = 16
NW = NC * NS
CH = 128
L = 16


def _gather_call(ids2d, wordtab, n_per_w, n_ch):
    mesh = plsc.VectorSubcoreMesh(core_axis_name="c", subcore_axis_name="s")
    N = NW * n_per_w

    @functools.partial(
        pl.kernel,
        mesh=mesh,
        out_type=jax.ShapeDtypeStruct((N, 32), jnp.int32),
        scratch_types=[
            pltpu.VMEM((n_ch, CH), jnp.int32),
            pltpu.VMEM((n_ch, CH), jnp.int32),
            pltpu.VMEM((n_per_w, 128), jnp.int32),
            pltpu.VMEM((n_per_w, 32), jnp.int32),
            pltpu.SemaphoreType.DMA,
        ],
        compiler_params=pltpu.CompilerParams(
            use_tc_tiling_on_sc=False, needs_layout_passes=False
        ),
    )
    def gather_kernel(ids_hbm, tab_hbm, out_hbm, idx_v, q_v, rows_v, sel_v, sem):
        wid = lax.axis_index("s") * NC + lax.axis_index("c")
        base = wid * n_per_w
        pltpu.sync_copy(ids_hbm.at[pl.ds(wid * n_ch, n_ch)], idx_v)

        def qbody(i, _):
            j, k = i // (CH // L), (i % (CH // L)) * L
            vec = idx_v[j, pl.ds(k, L)]
            q_v[j, pl.ds(k, L)] = lax.shift_right_logical(vec, 2)
            return 0

        lax.fori_loop(0, n_ch * (CH // L), qbody, 0, unroll=True)

        copies = []
        for j in range(n_ch):
            copies.append(
                pltpu.async_copy(
                    tab_hbm.at[q_v.at[j]],
                    rows_v.at[pl.ds(j * CH, CH)],
                    sem,
                )
            )
        for c in copies:
            c.wait()

        lanes = lax.broadcasted_iota(jnp.int32, (L,), 0)

        def sbody(g, _):
            j, k0 = g // (CH // L), (g % (CH // L)) * L
            idxv = idx_v[j, pl.ds(k0, L)]
            rowv = j * CH + k0 + lanes
            col0 = lax.shift_left(lax.bitwise_and(idxv, 3), 5)

            def wbody(k, _):
                w = plsc.load_gather(rows_v, [rowv, col0 + k])
                plsc.store_scatter(sel_v, [rowv, lax.full((L,), 0, jnp.int32) + k], w)
                return 0

            lax.fori_loop(0, 32, wbody, 0, unroll=8)
            return 0

        lax.fori_loop(0, n_per_w // L, sbody, 0)
        pltpu.sync_copy(sel_v, out_hbm.at[pl.ds(base, n_per_w)])

    return gather_kernel(ids2d, wordtab)


def kernel(input_ids, wte):
    B, S = input_ids.shape
    V, D = wte.shape
    N = B * S
    n_per_w = N // NW
    n_ch = n_per_w // CH
    ids2d = input_ids.reshape(NW * n_ch, CH)
    wordtab = _format_words(wte.T)
    out32 = _gather_call(ids2d, wordtab, n_per_w, n_ch)
    out = lax.bitcast_convert_type(out32, jnp.bfloat16).reshape(B, S, D)
    return out.astype(jnp.float32)


# speedup vs baseline: 45.3907x; 1.0169x over previous
import functools

import jax
import jax.numpy as jnp
from jax import lax
from jax.experimental import pallas as pl
from jax.experimental.pallas import tpu as pltpu
from jax.experimental.pallas import tpu_sc as plsc


def _format_words(wte_t):
    D, V = wte_t.shape
    blk = 8192
    grid = (V + blk - 1) // blk
    nq = blk * D // 256

    dk = D // 2

    def body(in_ref, out_ref):
        ks = lax.broadcasted_iota(jnp.int32, (D, dk), 1)
        ds_ = lax.broadcasted_iota(jnp.int32, (D, dk), 0)
        p_even = (ds_ == 2 * ks).astype(jnp.bfloat16)
        p_odd = (ds_ == 2 * ks + 1).astype(jnp.bfloat16)
        x = in_ref[...]
        dn = (((0,), (0,)), ((), ()))
        ye = lax.dot_general(x, p_even, dn, preferred_element_type=jnp.float32)
        yo = lax.dot_general(x, p_odd, dn, preferred_element_type=jnp.float32)
        be = lax.bitcast_convert_type(ye, jnp.int32)
        bo = lax.bitcast_convert_type(yo, jnp.int32)
        w = lax.bitwise_or(
            lax.shift_right_logical(be, 16),
            lax.bitwise_and(bo, jnp.int32(-65536)),
        )
        w4 = w.reshape(nq, 4, dk)
        out_ref[...] = jnp.concatenate(
            [w4[:, a, :] for a in range(4)], axis=1
        )

    return pl.pallas_call(
        body,
        grid=(grid,),
        in_specs=[pl.BlockSpec((D, blk), lambda i: (0, i))],
        out_specs=pl.BlockSpec((nq, 128), lambda i: (i, 0)),
        out_shape=jax.ShapeDtypeStruct((V * D // 256, 128), jnp.int32),
    )(wte_t)

NC = 2
NS = 16
NW = NC * NS
CH = 128
L = 16


def _gather_call(ids2d, wordtab, n_per_w, n_ch):
    mesh = plsc.VectorSubcoreMesh(core_axis_name="c", subcore_axis_name="s")
    N = NW * n_per_w

    @functools.partial(
        pl.kernel,
        mesh=mesh,
        out_type=jax.ShapeDtypeStruct((N, 64), jnp.float32),
        scratch_types=[
            pltpu.VMEM((n_ch, CH), jnp.int32),
            pltpu.VMEM((n_ch, CH), jnp.int32),
            pltpu.VMEM((n_per_w, 128), jnp.int32),
            pltpu.VMEM((n_per_w, 64), jnp.float32),
            pltpu.SemaphoreType.DMA,
        ],
        compiler_params=pltpu.CompilerParams(
            use_tc_tiling_on_sc=False, needs_layout_passes=False
        ),
    )
    def gather_kernel(ids_hbm, tab_hbm, out_hbm, idx_v, q_v, rows_v, sel_v, sem):
        wid = lax.axis_index("s") * NC + lax.axis_index("c")
        base = wid * n_per_w
        pltpu.sync_copy(ids_hbm.at[pl.ds(wid * n_ch, n_ch)], idx_v)

        def qbody(i, _):
            j, k = i // (CH // L), (i % (CH // L)) * L
            vec = idx_v[j, pl.ds(k, L)]
            q_v[j, pl.ds(k, L)] = lax.shift_right_logical(vec, 2)
            return 0

        lax.fori_loop(0, n_ch * (CH // L), qbody, 0, unroll=True)

        copies = []
        for j in range(n_ch):
            copies.append(
                pltpu.async_copy(
                    tab_hbm.at[q_v.at[j]],
                    rows_v.at[pl.ds(j * CH, CH)],
                    sem,
                )
            )
        for c in copies:
            c.wait()

        lanes = lax.broadcasted_iota(jnp.int32, (L,), 0)

        def sbody(g, _):
            j, k0 = g // (CH // L), (g % (CH // L)) * L
            idxv = idx_v[j, pl.ds(k0, L)]
            rowv = j * CH + k0 + lanes
            col0 = lax.shift_left(lax.bitwise_and(idxv, 3), 5)

            def wbody(k, _):
                w = plsc.load_gather(rows_v, [rowv, col0 + k])
                lo = plsc.bitcast(lax.shift_left(w, 16), jnp.float32)
                hi = plsc.bitcast(
                    lax.bitwise_and(w, jnp.int32(-65536)), jnp.float32
                )
                zeros = lax.full((L,), 0, jnp.int32)
                plsc.store_scatter(sel_v, [rowv, zeros + 2 * k], lo)
                plsc.store_scatter(sel_v, [rowv, zeros + 2 * k + 1], hi)
                return 0

            lax.fori_loop(0, 32, wbody, 0, unroll=8)
            return 0

        lax.fori_loop(0, n_per_w // L, sbody, 0)
        pltpu.sync_copy(sel_v, out_hbm.at[pl.ds(base, n_per_w)])

    return gather_kernel(ids2d, wordtab)


def kernel(input_ids, wte):
    B, S = input_ids.shape
    V, D = wte.shape
    N = B * S
    n_per_w = N // NW
    n_ch = n_per_w // CH
    ids2d = input_ids.reshape(NW * n_ch, CH)
    wordtab = _format_words(wte.T)
    out = _gather_call(ids2d, wordtab, n_per_w, n_ch)
    return out.reshape(B, S, D)
